# baseline (speedup 1.0000x reference)
import functools

import jax
import jax.numpy as jnp
from jax import lax
from jax.experimental import pallas as pl
from jax.experimental.pallas import tpu as pltpu
from jax.experimental.pallas import tpu_sc as plsc

NC = 2
NS = 16
NW = NC * NS

CHUNK = 128
NBUF = 10

VOCAB = 1000000
DP = 64
NVREG = DP // 16
SEG_LEN = 1000
BATCH = 1024
SEG_PER_W = BATCH // NW
IDX_PER_W = SEG_PER_W * SEG_LEN
CHUNKS_PER_W = IDX_PER_W // CHUNK

ROWS_BM = 5000


def _proj_body(xa_ref, xb_ref, w_ref, b_ref, o_ref):
  pa = lax.dot_general(
      xa_ref[...], w_ref[...], (((1,), (1,)), ((), ())),
      preferred_element_type=jnp.float32) + b_ref[...]
  pb = lax.dot_general(
      xb_ref[...], w_ref[...], (((1,), (1,)), ((), ())),
      preferred_element_type=jnp.float32) + b_ref[...]
  o_ref[...] = jnp.concatenate([pa, pb], axis=1)


def _sc_body(idx_hbm, p_hbm, out_hbm, idx_v, rows_bufs, stag, sems):
  wid = lax.axis_index("s") * NC + lax.axis_index("c")
  base = wid * IDX_PER_W

  pltpu.sync_copy(idx_hbm.at[pl.ds(base, IDX_PER_W)], idx_v)

  def start_gather(g, b):
    pltpu.async_copy(
        p_hbm.at[idx_v.at[pl.ds(g * CHUNK, CHUNK)]], rows_bufs[b], sems[b])

  def wait_gather(b):
    pltpu.make_async_copy(
        p_hbm.at[idx_v.at[pl.ds(0, CHUNK)]], rows_bufs[b], sems[b]).wait()

  for b in range(NBUF):
    start_gather(b, b)

  zeros = tuple(jnp.zeros((16,), jnp.float32) for _ in range(NVREG))
  inv = jnp.float32(1.0 / SEG_LEN)

  def accum_pairs(rows, plo, phi, acc):
    def pair_body(q, acc):
      r0 = 2 * q
      new = []
      for k in range(NVREG):
        v = rows[r0, pl.ds(16 * k, 16)] + rows[r0 + 1, pl.ds(16 * k, 16)]
        new.append(acc[k] + v)
      return tuple(new)

    return lax.fori_loop(plo, phi, pair_body, acc)

  def process_chunk(g, b, acc):
    rows = rows_bufs[b]
    seg = (g * CHUNK) // SEG_LEN
    bnd = (seg + 1) * SEG_LEN - g * CHUNK
    has_b = bnd <= CHUNK
    nb = jnp.minimum(bnd, CHUNK)

    acc = accum_pairs(rows, 0, nb // 2, acc)

    @pl.when(has_b)
    def _flush():
      for k in range(NVREG):
        stag[seg, pl.ds(16 * k, 16)] = acc[k] * inv

    keep = jnp.where(has_b, jnp.float32(0.0), jnp.float32(1.0))
    acc = tuple(a * keep for a in acc)
    acc = accum_pairs(rows, nb // 2, CHUNK // 2, acc)
    return acc

  def outer(o, acc):
    for b in range(NBUF):
      g = o * NBUF + b
      wait_gather(b)
      acc = process_chunk(g, b, acc)

      @pl.when(g + NBUF < CHUNKS_PER_W)
      def _next():
        start_gather(g + NBUF, b)

    return acc

  lax.fori_loop(0, CHUNKS_PER_W // NBUF, outer, zeros)

  pltpu.sync_copy(stag, out_hbm.at[pl.ds(wid * SEG_PER_W, SEG_PER_W)])


@jax.jit
def kernel(sents_batch, emb_table, W, b):
  batch, sent_count, sent_len = sents_batch.shape
  n_class = W.shape[0]
  flat_idx = sents_batch.reshape(batch * sent_count * sent_len)
  flat_idx = jnp.where(flat_idx < VOCAB // 2,
                       2 * flat_idx,
                       2 * flat_idx - (VOCAB - 1)).astype(jnp.int32)

  w_pad = jnp.zeros((DP, W.shape[1]), jnp.float32).at[:n_class].set(W)
  b_pad = jnp.zeros((1, DP), jnp.float32).at[0, :n_class].set(b)

  half_blocks = (VOCAB // 2) // ROWS_BM
  proj = pl.pallas_call(
      _proj_body,
      grid=(half_blocks,),
      in_specs=[
          pl.BlockSpec((ROWS_BM, emb_table.shape[1]), lambda i: (i, 0)),
          pl.BlockSpec((ROWS_BM, emb_table.shape[1]),
                       lambda i: (i + half_blocks, 0)),
          pl.BlockSpec((DP, emb_table.shape[1]), lambda i: (0, 0)),
          pl.BlockSpec((1, DP), lambda i: (0, 0)),
      ],
      out_specs=pl.BlockSpec((ROWS_BM, 2 * DP), lambda i: (i, 0)),
      out_shape=jax.ShapeDtypeStruct((VOCAB // 2, 2 * DP), jnp.float32),
  )(emb_table, emb_table, w_pad, b_pad)
  proj = proj.reshape(VOCAB, DP)

  mesh = plsc.VectorSubcoreMesh(
      core_axis_name="c", subcore_axis_name="s",
      num_cores=NC, num_subcores=NS)

  sc_fn = pl.kernel(
      _sc_body,
      out_type=jax.ShapeDtypeStruct((BATCH, DP), jnp.float32),
      mesh=mesh,
      compiler_params=pltpu.CompilerParams(use_tc_tiling_on_sc=False),
      scratch_types=[
          pltpu.VMEM((IDX_PER_W,), jnp.int32),
          [pltpu.VMEM((CHUNK, DP), jnp.float32) for _ in range(NBUF)],
          pltpu.VMEM((SEG_PER_W, DP), jnp.float32),
          [pltpu.SemaphoreType.DMA for _ in range(NBUF)],
      ],
  )
  pooled = sc_fn(flat_idx, proj)
  return pooled[:, :n_class]


# speedup vs baseline: 1.0013x; 1.0013x over previous
import functools

import jax
import jax.numpy as jnp
from jax import lax
from jax.experimental import pallas as pl
from jax.experimental.pallas import tpu as pltpu
from jax.experimental.pallas import tpu_sc as plsc

NC = 2
NS = 16
NW = NC * NS

CHUNK = 128
NBUF = 5

VOCAB = 1000000
DP = 64
NVREG = DP // 16
SEG_LEN = 1000
BATCH = 1024
SEG_PER_W = BATCH // NW
IDX_PER_W = SEG_PER_W * SEG_LEN
CHUNKS_PER_W = IDX_PER_W // CHUNK

ROWS_BM = 5000


def _proj_body(xa_ref, xb_ref, w_ref, b_ref, o_ref):
  pa = lax.dot_general(
      xa_ref[...], w_ref[...], (((1,), (1,)), ((), ())),
      preferred_element_type=jnp.float32) + b_ref[...]
  pb = lax.dot_general(
      xb_ref[...], w_ref[...], (((1,), (1,)), ((), ())),
      preferred_element_type=jnp.float32) + b_ref[...]
  o_ref[...] = jnp.concatenate([pa, pb], axis=1)


def _sc_body(idx_hbm, p_hbm, out_hbm, idx_v, rows_bufs, stag, sems):
  wid = lax.axis_index("s") * NC + lax.axis_index("c")
  base = wid * IDX_PER_W

  pltpu.sync_copy(idx_hbm.at[pl.ds(base, IDX_PER_W)], idx_v)

  def start_gather(g, b):
    pltpu.async_copy(
        p_hbm.at[idx_v.at[pl.ds(g * CHUNK, CHUNK)]], rows_bufs[b], sems[b])

  def wait_gather(b):
    pltpu.make_async_copy(
        p_hbm.at[idx_v.at[pl.ds(0, CHUNK)]], rows_bufs[b], sems[b]).wait()

  for b in range(NBUF):
    start_gather(b, b)

  zeros = tuple(jnp.zeros((16,), jnp.float32) for _ in range(NVREG))
  inv = jnp.float32(1.0 / SEG_LEN)

  def accum_pairs(rows, plo, phi, acc):
    def pair_body(q, acc):
      r0 = 2 * q
      new = []
      for k in range(NVREG):
        v = rows[r0, pl.ds(16 * k, 16)] + rows[r0 + 1, pl.ds(16 * k, 16)]
        new.append(acc[k] + v)
      return tuple(new)

    return lax.fori_loop(plo, phi, pair_body, acc)

  def process_chunk(g, b, acc):
    rows = rows_bufs[b]
    seg = (g * CHUNK) // SEG_LEN
    bnd = (seg + 1) * SEG_LEN - g * CHUNK
    has_b = bnd <= CHUNK
    nb = jnp.minimum(bnd, CHUNK)

    acc = accum_pairs(rows, 0, nb // 2, acc)

    @pl.when(has_b)
    def _flush():
      for k in range(NVREG):
        stag[seg, pl.ds(16 * k, 16)] = acc[k] * inv

    keep = jnp.where(has_b, jnp.float32(0.0), jnp.float32(1.0))
    acc = tuple(a * keep for a in acc)
    acc = accum_pairs(rows, nb // 2, CHUNK // 2, acc)
    return acc

  def outer(o, acc):
    for b in range(NBUF):
      g = o * NBUF + b
      wait_gather(b)
      acc = process_chunk(g, b, acc)

      @pl.when(g + NBUF < CHUNKS_PER_W)
      def _next():
        start_gather(g + NBUF, b)

    return acc

  lax.fori_loop(0, CHUNKS_PER_W // NBUF, outer, zeros)

  pltpu.sync_copy(stag, out_hbm.at[pl.ds(wid * SEG_PER_W, SEG_PER_W)])


@jax.jit
def kernel(sents_batch, emb_table, W, b):
  batch, sent_count, sent_len = sents_batch.shape
  n_class = W.shape[0]
  flat_idx = sents_batch.reshape(batch * sent_count * sent_len)
  flat_idx = jnp.where(flat_idx < VOCAB // 2,
                       2 * flat_idx,
                       2 * flat_idx - (VOCAB - 1)).astype(jnp.int32)

  w_pad = jnp.zeros((DP, W.shape[1]), jnp.float32).at[:n_class].set(W)
  b_pad = jnp.zeros((1, DP), jnp.float32).at[0, :n_class].set(b)

  half_blocks = (VOCAB // 2) // ROWS_BM
  proj = pl.pallas_call(
      _proj_body,
      grid=(half_blocks,),
      in_specs=[
          pl.BlockSpec((ROWS_BM, emb_table.shape[1]), lambda i: (i, 0)),
          pl.BlockSpec((ROWS_BM, emb_table.shape[1]),
                       lambda i: (i + half_blocks, 0)),
          pl.BlockSpec((DP, emb_table.shape[1]), lambda i: (0, 0)),
          pl.BlockSpec((1, DP), lambda i: (0, 0)),
      ],
      out_specs=pl.BlockSpec((ROWS_BM, 2 * DP), lambda i: (i, 0)),
      out_shape=jax.ShapeDtypeStruct((VOCAB // 2, 2 * DP), jnp.float32),
  )(emb_table, emb_table, w_pad, b_pad)
  proj = proj.reshape(VOCAB, DP)

  mesh = plsc.VectorSubcoreMesh(
      core_axis_name="c", subcore_axis_name="s",
      num_cores=NC, num_subcores=NS)

  sc_fn = pl.kernel(
      _sc_body,
      out_type=jax.ShapeDtypeStruct((BATCH, DP), jnp.float32),
      mesh=mesh,
      compiler_params=pltpu.CompilerParams(use_tc_tiling_on_sc=False),
      scratch_types=[
          pltpu.VMEM((IDX_PER_W,), jnp.int32),
          [pltpu.VMEM((CHUNK, DP), jnp.float32) for _ in range(NBUF)],
          pltpu.VMEM((SEG_PER_W, DP), jnp.float32),
          [pltpu.SemaphoreType.DMA for _ in range(NBUF)],
      ],
  )
  pooled = sc_fn(flat_idx, proj)
  return pooled[:, :n_class]
